# Initial kernel scaffold; baseline (speedup 1.0000x reference)
import jax
import jax.numpy as jnp
from jax.experimental import pallas as pl


def kernel(subs, table):
    raise NotImplementedError("write your pallas kernel here")


# speedup vs baseline: 37.7906x; 37.7906x over previous
import functools

import jax
import jax.numpy as jnp
from jax import lax
from jax.experimental import pallas as pl
from jax.experimental.pallas import tpu as pltpu
from jax.experimental.pallas import tpu_sc as plsc

B = 4096
L = 200
D = 64
VOCAB = 1001
VPAD = 1024

NW = 32
TPW = B // NW
RT = 64
LANES = 16


def _sc_hist(subs_flat):
    mesh = plsc.VectorSubcoreMesh(core_axis_name="c", subcore_axis_name="s")
    info = plsc.get_sparse_core_info()
    nc = info.num_cores

    @functools.partial(
        pl.kernel,
        out_type=jax.ShapeDtypeStruct((B * VPAD,), jnp.float32),
        mesh=mesh,
        scratch_types=[
            pltpu.VMEM((RT * L,), jnp.int32),
            pltpu.VMEM((RT * VPAD,), jnp.float32),
        ],
        compiler_params=pltpu.CompilerParams(
            needs_layout_passes=False,
            use_tc_tiling_on_sc=False,
        ),
    )
    def hist_kernel(subs_hbm, c_hbm, subs_v, hist_v):
        wid = lax.axis_index("s") * nc + lax.axis_index("c")
        ones = jnp.full((LANES,), 1.0, jnp.float32)
        zeros = jnp.zeros((LANES,), jnp.float32)
        iota = lax.iota(jnp.int32, LANES)
        straddle = jnp.where(iota < 8, 0, VPAD)

        for r in range(B // (NW * RT)):
            tok_base = wid * TPW + r * RT
            pltpu.sync_copy(
                subs_hbm.at[pl.ds(tok_base * L, RT * L)], subs_v
            )

            def zero_body(i, _):
                for j in range(16):
                    hist_v[pl.ds((i * 16 + j) * LANES, LANES)] = zeros
                return 0

            lax.fori_loop(0, RT * VPAD // (16 * LANES), zero_body, 0)

            def pair_body(p, _):
                base0 = 2 * p * VPAD
                for c in range(2 * L // LANES):
                    svec = subs_v[pl.ds(p * 2 * L + c * LANES, LANES)]
                    if c * LANES + LANES <= L:
                        base = lax.broadcast(base0, (LANES,))
                    elif c * LANES >= L:
                        base = lax.broadcast(base0 + VPAD, (LANES,))
                    else:
                        base = lax.broadcast(base0, (LANES,)) + straddle
                    plsc.addupdate_scatter(hist_v, [base + svec], ones)
                return 0

            lax.fori_loop(0, RT // 2, pair_body, 0)

            pltpu.sync_copy(hist_v, c_hbm.at[pl.ds(tok_base * VPAD, RT * VPAD)])

    return hist_kernel(subs_flat)


def _tc_pool(hist, tbl_pad):
    blk = 256

    def body(c_ref, t_ref, o_ref):
        cvals = c_ref[...]
        acc = jnp.dot(cvals, t_ref[...], preferred_element_type=jnp.float32)
        cnt = float(L) - cvals[:, 0:1]
        o_ref[...] = acc / cnt

    return pl.pallas_call(
        body,
        grid=(B // blk,),
        in_specs=[
            pl.BlockSpec((blk, VPAD), lambda i: (i, 0)),
            pl.BlockSpec((VPAD, D), lambda i: (0, 0)),
        ],
        out_specs=pl.BlockSpec((blk, D), lambda i: (i, 0)),
        out_shape=jax.ShapeDtypeStruct((B, D), jnp.float32),
    )(hist, tbl_pad)


def kernel(subs, table):
    subs_flat = jnp.reshape(subs, (-1,)).astype(jnp.int32)
    hist = jnp.reshape(_sc_hist(subs_flat), (B, VPAD))
    tbl_pad = jnp.zeros((VPAD, D), jnp.float32).at[:VOCAB].set(table)
    return _tc_pool(hist, tbl_pad)


# speedup vs baseline: 45.4782x; 1.2034x over previous
import functools

import jax
import jax.numpy as jnp
from jax import lax
from jax.experimental import pallas as pl
from jax.experimental.pallas import tpu as pltpu
from jax.experimental.pallas import tpu_sc as plsc

B = 4096
L = 200
D = 64
VOCAB = 1001
VPAD = 1024
NJ = 8
BLK = 256

NW = 32
TPW = B // NW
RT = 64
LANES = 16


def _sc_hist(subs_flat):
    mesh = plsc.VectorSubcoreMesh(core_axis_name="c", subcore_axis_name="s")
    info = plsc.get_sparse_core_info()
    nc = info.num_cores

    @functools.partial(
        pl.kernel,
        out_type=jax.ShapeDtypeStruct((B // BLK, NJ, BLK, 128), jnp.float32),
        mesh=mesh,
        scratch_types=[
            pltpu.VMEM((RT * L,), jnp.int32),
            pltpu.VMEM((RT, VPAD), jnp.float32),
        ],
        compiler_params=pltpu.CompilerParams(
            needs_layout_passes=False,
            use_tc_tiling_on_sc=False,
        ),
    )
    def hist_kernel(subs_hbm, c_hbm, subs_v, hist_v):
        wid = lax.axis_index("s") * nc + lax.axis_index("c")
        blk_id = wid // 2
        half = (wid % 2) * TPW
        ones = jnp.full((LANES,), 1.0, jnp.float32)
        zeros = jnp.zeros((LANES,), jnp.float32)
        iota = lax.iota(jnp.int32, LANES)
        straddle = jnp.where(iota < 8, 0, 1)

        for r in range(TPW // RT):
            tok_base = wid * TPW + r * RT
            pltpu.sync_copy(
                subs_hbm.at[pl.ds(tok_base * L, RT * L)], subs_v
            )

            def zero_body(i, _):
                for j in range(VPAD // LANES):
                    hist_v[i, pl.ds(j * LANES, LANES)] = zeros
                return 0

            lax.fori_loop(0, RT, zero_body, 0)

            def pair_body(p, _):
                t0 = 2 * p
                for c in range(2 * L // LANES):
                    svec = subs_v[pl.ds(p * 2 * L + c * LANES, LANES)]
                    if c * LANES + LANES <= L:
                        row = lax.broadcast(t0, (LANES,))
                    elif c * LANES >= L:
                        row = lax.broadcast(t0 + 1, (LANES,))
                    else:
                        row = lax.broadcast(t0, (LANES,)) + straddle
                    plsc.addupdate_scatter(hist_v, [row, svec], ones)
                return 0

            lax.fori_loop(0, RT // 2, pair_body, 0)

            for j in range(NJ):
                pltpu.sync_copy(
                    hist_v.at[:, pl.ds(j * 128, 128)],
                    c_hbm.at[blk_id, j, pl.ds(half + r * RT, RT), :],
                )

    return hist_kernel(subs_flat)


def _tc_pool(hist4, tbl4):
    def body(c_ref, t_ref, o_ref):
        acc = jnp.dot(
            c_ref[0, 0], t_ref[0], preferred_element_type=jnp.float32
        )
        for j in range(1, NJ):
            acc += jnp.dot(
                c_ref[0, j], t_ref[j], preferred_element_type=jnp.float32
            )
        cnt = float(L) - c_ref[0, 0, :, 0:1]
        o_ref[...] = acc / cnt

    return pl.pallas_call(
        body,
        grid=(B // BLK,),
        in_specs=[
            pl.BlockSpec((1, NJ, BLK, 128), lambda i: (i, 0, 0, 0)),
            pl.BlockSpec((NJ, 128, D), lambda i: (0, 0, 0)),
        ],
        out_specs=pl.BlockSpec((BLK, D), lambda i: (i, 0)),
        out_shape=jax.ShapeDtypeStruct((B, D), jnp.float32),
    )(hist4, tbl4)


def kernel(subs, table):
    subs_flat = jnp.reshape(subs, (-1,)).astype(jnp.int32)
    hist4 = _sc_hist(subs_flat)
    tbl_pad = jnp.zeros((VPAD, D), jnp.float32).at[:VOCAB].set(table)
    return _tc_pool(hist4, jnp.reshape(tbl_pad, (NJ, 128, D)))
